# baseline (speedup 1.0000x reference)
import functools

import jax
import jax.numpy as jnp
from jax import lax
from jax.experimental import pallas as pl
from jax.experimental.pallas import tpu as pltpu
from jax.experimental.pallas import tpu_sc as plsc

_K1 = [21, 15, 9, 5]
_P1 = [10, 7, 4, 2]
_K2 = [9, 7, 5, 3]
_P2 = [4, 3, 2, 1]
_GROUPS = 4
_VOCAB = 8192
_D = 200
_DPAD = 256


def _conv(x, w, b, pad, groups=1):
    y = lax.conv_general_dilated(
        x, w, window_strides=(1, 1), padding=((0, 0), (pad, pad)),
        dimension_numbers=('NCHW', 'OIHW', 'NCHW'), feature_group_count=groups)
    return y + b[None, :, None, None]


def _gn(x, g, b, groups=_GROUPS, eps=1e-5):
    B, C, H, W = x.shape
    xg = x.reshape(B, groups, C // groups, H, W)
    mu = xg.mean(axis=(2, 3, 4), keepdims=True)
    var = xg.var(axis=(2, 3, 4), keepdims=True)
    xg = (xg - mu) / jnp.sqrt(var + eps)
    xn = xg.reshape(B, C, H, W)
    return xn * g[None, :, None, None] + b[None, :, None, None]


def _pool(x, k):
    B, C, H, W = x.shape
    return x.reshape(B, C, H, W // k, k).mean(axis=-1)


def _merged_encoder(h, p):
    C = 8
    w1 = jnp.zeros((4 * C, 1, 1, _K1[0]), jnp.float32)
    b1 = jnp.concatenate([p['c1b'][i] for i in range(4)])
    for i in range(4):
        o = _P1[0] - _P1[i]
        w1 = w1.at[i * C:(i + 1) * C, :, :, o:o + _K1[i]].set(p['c1w'][i])
    g1 = jnp.concatenate([p['g1w'][i] for i in range(4)])
    gb1 = jnp.concatenate([p['g1b'][i] for i in range(4)])
    w2 = jnp.zeros((4 * C, C, 1, _K2[0]), jnp.float32)
    b2 = jnp.concatenate([p['c2b'][i] for i in range(4)])
    for i in range(4):
        o = _P2[0] - _P2[i]
        w2 = w2.at[i * C:(i + 1) * C, :, :, o:o + _K2[i]].set(p['c2w'][i])
    g2 = jnp.concatenate([p['g2w'][i] for i in range(4)])
    gb2 = jnp.concatenate([p['g2b'][i] for i in range(4)])

    y = _pool(jax.nn.gelu(_gn(_conv(h, w1, b1, _P1[0]), g1, gb1,
                              groups=4 * _GROUPS), approximate=False), 2)
    y = _pool(jax.nn.gelu(_gn(_conv(y, w2, b2, _P2[0], groups=4), g2, gb2,
                              groups=4 * _GROUPS), approximate=False), 4)
    B, C4, NA, T = y.shape
    z = jnp.transpose(y.reshape(B, 4, C, NA, T), (1, 0, 3, 4, 2))
    return z.reshape(4 * B * NA, T * C), (B, NA, T * C)


def _argmin_body(r_ref, cb_ref, rn_ref, cn_ref, out_ref):
    r = r_ref[...]
    cb = cb_ref[...]
    m = lax.dot_general(r, cb, (((1,), (1,)), ((), ())),
                        precision=lax.Precision.DEFAULT,
                        preferred_element_type=jnp.float32)
    d = (rn_ref[...] - 2.0 * m) + cn_ref[...]
    dmin = jnp.min(d, axis=1, keepdims=True)
    ids = lax.broadcasted_iota(jnp.int32, d.shape, 1)
    idx = jnp.min(jnp.where(d == dmin, ids, d.shape[1]), axis=1)
    out_ref[...] = idx[:, None]


def _nearest_code(r, cb, rn, cn, tm=256):
    M, D = r.shape
    K = cb.shape[0]
    return pl.pallas_call(
        _argmin_body,
        grid=(M // tm,),
        in_specs=[
            pl.BlockSpec((tm, D), lambda t: (t, 0)),
            pl.BlockSpec((K, D), lambda t: (0, 0)),
            pl.BlockSpec((tm, 1), lambda t: (t, 0)),
            pl.BlockSpec((1, K), lambda t: (0, 0)),
        ],
        out_specs=pl.BlockSpec((tm, 1), lambda t: (t, 0)),
        out_shape=jax.ShapeDtypeStruct((M, 1), jnp.int32),
    )(r, cb, rn, cn)


def _pad_body(x_ref, out_ref):
    x = x_ref[...]
    out_ref[...] = jnp.pad(x, ((0, 0), (0, _DPAD - _D)))


def _pad_table(cbs_flat, rows_per_blk=1024):
    R = cbs_flat.shape[0]
    return pl.pallas_call(
        _pad_body,
        grid=(R // rows_per_blk,),
        in_specs=[pl.BlockSpec((rows_per_blk, _D), lambda t: (t, 0))],
        out_specs=pl.BlockSpec((rows_per_blk, _DPAD), lambda t: (t, 0)),
        out_shape=jax.ShapeDtypeStruct((R, _DPAD), jnp.float32),
    )(cbs_flat)


def _sc_gather(table, idx2d):
    V, Dp = table.shape
    B = idx2d.shape[0] * idx2d.shape[1]
    info = plsc.get_sparse_core_info()
    nw = info.num_cores * info.num_subcores
    b_per_w = B // nw
    chunks = b_per_w // 128
    mesh = plsc.VectorSubcoreMesh(core_axis_name="c", subcore_axis_name="s")

    @functools.partial(
        pl.kernel, mesh=mesh,
        out_type=jax.ShapeDtypeStruct((B, Dp), jnp.float32),
        scratch_types=[
            pltpu.VMEM((chunks, 128), jnp.int32),
            pltpu.VMEM((b_per_w, Dp), jnp.float32),
            pltpu.SemaphoreType.DMA,
        ],
    )
    def gk(table_hbm, idx_hbm, out_hbm, idx_v, rows_v, sem):
        wid = lax.axis_index("s") * info.num_cores + lax.axis_index("c")
        base = wid * b_per_w
        pltpu.sync_copy(idx_hbm.at[pl.ds(wid * chunks, chunks)], idx_v)
        cps = [pltpu.async_copy(table_hbm.at[idx_v.at[j]],
                                rows_v.at[pl.ds(j * 128, 128)], sem)
               for j in range(chunks)]
        for c in cps:
            c.wait()
        pltpu.sync_copy(rows_v, out_hbm.at[pl.ds(base, b_per_w)])

    return gk(table, idx2d)


def kernel(x, params):
    B, N, A, T = x.shape
    h = x.reshape(B, N * A, T)[:, None, :, :]
    zf, z_shape = _merged_encoder(h, params)

    cbs = params['codebooks']
    n_q = cbs.shape[0]
    table = _pad_table(cbs.reshape(n_q * _VOCAB, _D))

    total = jnp.zeros_like(zf)
    residual = zf
    for l in range(n_q):
        cb = cbs[l]
        cn = (cb ** 2).sum(-1)[None, :]
        rn = (residual ** 2).sum(-1, keepdims=True)
        idx = _nearest_code(residual, cb, rn, cn)
        gidx = idx.reshape(-1, 128) + (l * _VOCAB)
        q = _sc_gather(table, gidx)[:, :_D]
        total = total + q
        residual = residual - q

    q_st = zf + (total - zf)
    return q_st.reshape(4, z_shape[0], z_shape[1], z_shape[2])


# speedup vs baseline: 1.4322x; 1.4322x over previous
import functools

import jax
import jax.numpy as jnp
from jax import lax
from jax.experimental import pallas as pl
from jax.experimental.pallas import tpu as pltpu
from jax.experimental.pallas import tpu_sc as plsc

_K1 = [21, 15, 9, 5]
_P1 = [10, 7, 4, 2]
_K2 = [9, 7, 5, 3]
_P2 = [4, 3, 2, 1]
_GROUPS = 4
_VOCAB = 8192
_D = 200
_DPAD = 256


def _conv(x, w, b, pad, groups=1):
    y = lax.conv_general_dilated(
        x, w, window_strides=(1, 1), padding=((0, 0), (pad, pad)),
        dimension_numbers=('NCHW', 'OIHW', 'NCHW'), feature_group_count=groups)
    return y + b[None, :, None, None]


def _gn(x, g, b, groups=_GROUPS, eps=1e-5):
    B, C, H, W = x.shape
    xg = x.reshape(B, groups, C // groups, H, W)
    mu = xg.mean(axis=(2, 3, 4), keepdims=True)
    var = xg.var(axis=(2, 3, 4), keepdims=True)
    xg = (xg - mu) / jnp.sqrt(var + eps)
    xn = xg.reshape(B, C, H, W)
    return xn * g[None, :, None, None] + b[None, :, None, None]


def _pool(x, k):
    B, C, H, W = x.shape
    return x.reshape(B, C, H, W // k, k).mean(axis=-1)


def _merged_encoder(h, p):
    C = 8
    w1 = jnp.zeros((4 * C, 1, 1, _K1[0]), jnp.float32)
    b1 = jnp.concatenate([p['c1b'][i] for i in range(4)])
    for i in range(4):
        o = _P1[0] - _P1[i]
        w1 = w1.at[i * C:(i + 1) * C, :, :, o:o + _K1[i]].set(p['c1w'][i])
    g1 = jnp.concatenate([p['g1w'][i] for i in range(4)])
    gb1 = jnp.concatenate([p['g1b'][i] for i in range(4)])
    w2 = jnp.zeros((4 * C, 4 * C, 1, _K2[0]), jnp.float32)
    b2 = jnp.concatenate([p['c2b'][i] for i in range(4)])
    for i in range(4):
        o = _P2[0] - _P2[i]
        w2 = w2.at[i * C:(i + 1) * C, i * C:(i + 1) * C, :, o:o + _K2[i]].set(
            p['c2w'][i])
    g2 = jnp.concatenate([p['g2w'][i] for i in range(4)])
    gb2 = jnp.concatenate([p['g2b'][i] for i in range(4)])

    y = _pool(jax.nn.gelu(_gn(_conv(h, w1, b1, _P1[0]), g1, gb1,
                              groups=4 * _GROUPS), approximate=False), 2)
    y = _pool(jax.nn.gelu(_gn(_conv(y, w2, b2, _P2[0]), g2, gb2,
                              groups=4 * _GROUPS), approximate=False), 4)
    B, C4, NA, T = y.shape
    z = jnp.transpose(y.reshape(B, 4, C, NA, T), (1, 0, 3, 4, 2))
    return z.reshape(4 * B * NA, T * C), (B, NA, T * C)


def _argmin_body(r_ref, cb_ref, rn_ref, cn_ref, out_ref):
    r = r_ref[...]
    cb = cb_ref[...]
    m = lax.dot_general(r, cb, (((1,), (1,)), ((), ())),
                        precision=lax.Precision.DEFAULT,
                        preferred_element_type=jnp.float32)
    d = (rn_ref[...] - 2.0 * m) + cn_ref[...]
    dmin = jnp.min(d, axis=1, keepdims=True)
    ids = lax.broadcasted_iota(jnp.int32, d.shape, 1)
    idx = jnp.min(jnp.where(d == dmin, ids, d.shape[1]), axis=1)
    out_ref[...] = idx[:, None]


def _nearest_code(r, cb, rn, cn, tm=256):
    M, D = r.shape
    K = cb.shape[0]
    return pl.pallas_call(
        _argmin_body,
        grid=(M // tm,),
        in_specs=[
            pl.BlockSpec((tm, D), lambda t: (t, 0)),
            pl.BlockSpec((K, D), lambda t: (0, 0)),
            pl.BlockSpec((tm, 1), lambda t: (t, 0)),
            pl.BlockSpec((1, K), lambda t: (0, 0)),
        ],
        out_specs=pl.BlockSpec((tm, 1), lambda t: (t, 0)),
        out_shape=jax.ShapeDtypeStruct((M, 1), jnp.int32),
    )(r, cb, rn, cn)


def _pad_body(x_ref, out_ref):
    x = x_ref[...]
    out_ref[...] = jnp.pad(x, ((0, 0), (0, _DPAD - _D)))


def _pad_table(cbs_flat, rows_per_blk=1024):
    R = cbs_flat.shape[0]
    return pl.pallas_call(
        _pad_body,
        grid=(R // rows_per_blk,),
        in_specs=[pl.BlockSpec((rows_per_blk, _D), lambda t: (t, 0))],
        out_specs=pl.BlockSpec((rows_per_blk, _DPAD), lambda t: (t, 0)),
        out_shape=jax.ShapeDtypeStruct((R, _DPAD), jnp.float32),
    )(cbs_flat)


def _sc_gather(table, idx2d):
    V, Dp = table.shape
    B = idx2d.shape[0] * idx2d.shape[1]
    info = plsc.get_sparse_core_info()
    nw = info.num_cores * info.num_subcores
    b_per_w = B // nw
    chunks = b_per_w // 128
    mesh = plsc.VectorSubcoreMesh(core_axis_name="c", subcore_axis_name="s")

    @functools.partial(
        pl.kernel, mesh=mesh,
        out_type=jax.ShapeDtypeStruct((B, Dp), jnp.float32),
        scratch_types=[
            pltpu.VMEM((chunks, 128), jnp.int32),
            pltpu.VMEM((b_per_w, Dp), jnp.float32),
            pltpu.SemaphoreType.DMA,
        ],
    )
    def gk(table_hbm, idx_hbm, out_hbm, idx_v, rows_v, sem):
        wid = lax.axis_index("s") * info.num_cores + lax.axis_index("c")
        base = wid * b_per_w
        pltpu.sync_copy(idx_hbm.at[pl.ds(wid * chunks, chunks)], idx_v)
        cps = [pltpu.async_copy(table_hbm.at[idx_v.at[j]],
                                rows_v.at[pl.ds(j * 128, 128)], sem)
               for j in range(chunks)]
        for c in cps:
            c.wait()
        pltpu.sync_copy(rows_v, out_hbm.at[pl.ds(base, b_per_w)])

    return gk(table, idx2d)


def kernel(x, params):
    B, N, A, T = x.shape
    h = x.reshape(B, N * A, T)[:, None, :, :]
    zf, z_shape = _merged_encoder(h, params)

    cbs = params['codebooks']
    n_q = cbs.shape[0]
    table = _pad_table(cbs.reshape(n_q * _VOCAB, _D))

    total = jnp.zeros_like(zf)
    residual = zf
    for l in range(n_q):
        cb = cbs[l]
        cn = (cb ** 2).sum(-1)[None, :]
        rn = (residual ** 2).sum(-1, keepdims=True)
        idx = _nearest_code(residual, cb, rn, cn)
        gidx = idx.reshape(-1, 128) + (l * _VOCAB)
        q = _sc_gather(table, gidx)[:, :_D]
        total = total + q
        residual = residual - q

    q_st = zf + (total - zf)
    return q_st.reshape(4, z_shape[0], z_shape[1], z_shape[2])


# speedup vs baseline: 1.8259x; 1.2749x over previous
import functools

import jax
import jax.numpy as jnp
from jax import lax
from jax.experimental import pallas as pl
from jax.experimental.pallas import tpu as pltpu
from jax.experimental.pallas import tpu_sc as plsc

_K1 = [21, 15, 9, 5]
_P1 = [10, 7, 4, 2]
_K2 = [9, 7, 5, 3]
_P2 = [4, 3, 2, 1]
_GROUPS = 4
_VOCAB = 8192
_D = 200
_DPAD = 256


def _conv(x, w, b, pad, groups=1):
    y = lax.conv_general_dilated(
        x, w, window_strides=(1, 1), padding=((0, 0), (pad, pad)),
        dimension_numbers=('NCHW', 'OIHW', 'NCHW'), feature_group_count=groups)
    return y + b[None, :, None, None]


def _gn(x, g, b, groups=_GROUPS, eps=1e-5):
    B, C, H, W = x.shape
    xg = x.reshape(B, groups, C // groups, H, W)
    mu = xg.mean(axis=(2, 3, 4), keepdims=True)
    var = xg.var(axis=(2, 3, 4), keepdims=True)
    xg = (xg - mu) / jnp.sqrt(var + eps)
    xn = xg.reshape(B, C, H, W)
    return xn * g[None, :, None, None] + b[None, :, None, None]


def _pool(x, k):
    B, C, H, W = x.shape
    return x.reshape(B, C, H, W // k, k).mean(axis=-1)


def _gn_nhwc(y, g, b, groups, eps=1e-5):
    B, NA, C, T = y.shape
    yg = y.reshape(B, NA, groups, C // groups, T)
    mu = yg.mean(axis=(1, 3, 4), keepdims=True)
    var = yg.var(axis=(1, 3, 4), keepdims=True)
    yn = ((yg - mu) / jnp.sqrt(var + eps)).reshape(B, NA, C, T)
    return yn * g[None, None, :, None] + b[None, None, :, None]


def _merged_encoder(h, p):
    C = 8
    B, _, NA, T = h.shape
    K, P = _K1[0], _P1[0]
    w1 = jnp.zeros((4 * C, K), jnp.float32)
    for i in range(4):
        o = P - _P1[i]
        w1 = w1.at[i * C:(i + 1) * C, o:o + _K1[i]].set(p['c1w'][i][:, 0, 0, :])
    b1 = jnp.concatenate([p['c1b'][i] for i in range(4)])
    jj = lax.broadcasted_iota(jnp.int32, (K, T + 2 * P, T), 1)
    tt = lax.broadcasted_iota(jnp.int32, (K, T + 2 * P, T), 2)
    dd = lax.broadcasted_iota(jnp.int32, (K, T + 2 * P, T), 0)
    band = (jj - tt == dd).astype(jnp.float32)
    toep1 = jnp.einsum('od,djt->ojt', w1, band)
    xp = jnp.pad(h[:, 0], ((0, 0), (0, 0), (P, P)))
    y = jnp.einsum('bnj,ojt->bnot', xp, toep1) + b1[None, None, :, None]
    g1 = jnp.concatenate([p['g1w'][i] for i in range(4)])
    gb1 = jnp.concatenate([p['g1b'][i] for i in range(4)])
    y = jax.nn.gelu(_gn_nhwc(y, g1, gb1, 4 * _GROUPS), approximate=False)
    T2 = T // 2
    y = y.reshape(B, NA, 4 * C, T2, 2).mean(axis=-1)

    K2, P2 = _K2[0], _P2[0]
    w2 = jnp.zeros((4 * C, 4 * C, K2), jnp.float32)
    for i in range(4):
        o = P2 - _P2[i]
        w2 = w2.at[i * C:(i + 1) * C, i * C:(i + 1) * C, o:o + _K2[i]].set(
            p['c2w'][i][:, :, 0, :])
    b2 = jnp.concatenate([p['c2b'][i] for i in range(4)])
    yp = jnp.pad(y, ((0, 0), (0, 0), (0, 0), (P2, P2)))
    y2 = b2[None, None, :, None]
    for k in range(K2):
        y2 = y2 + jnp.einsum('bnct,oc->bnot', yp[..., k:k + T2], w2[:, :, k])
    g2 = jnp.concatenate([p['g2w'][i] for i in range(4)])
    gb2 = jnp.concatenate([p['g2b'][i] for i in range(4)])
    y2 = jax.nn.gelu(_gn_nhwc(y2, g2, gb2, 4 * _GROUPS), approximate=False)
    T4 = T2 // 4
    y2 = y2.reshape(B, NA, 4 * C, T4, 4).mean(axis=-1)

    z = jnp.transpose(y2.reshape(B, NA, 4, C, T4), (2, 0, 1, 4, 3))
    return z.reshape(4 * B * NA, T4 * C), (B, NA, T4 * C)


def _argmin_body(r_ref, cb_ref, rn_ref, cn_ref, out_ref):
    r = r_ref[...]
    cb = cb_ref[...]
    m = lax.dot_general(r, cb, (((1,), (1,)), ((), ())),
                        precision=lax.Precision.DEFAULT,
                        preferred_element_type=jnp.float32)
    d = (rn_ref[...] - 2.0 * m) + cn_ref[...]
    dmin = jnp.min(d, axis=1, keepdims=True)
    ids = lax.broadcasted_iota(jnp.int32, d.shape, 1)
    idx = jnp.min(jnp.where(d == dmin, ids, d.shape[1]), axis=1)
    out_ref[...] = idx[:, None]


def _nearest_code(r, cb, rn, cn, tm=256):
    M, D = r.shape
    K = cb.shape[0]
    return pl.pallas_call(
        _argmin_body,
        grid=(M // tm,),
        in_specs=[
            pl.BlockSpec((tm, D), lambda t: (t, 0)),
            pl.BlockSpec((K, D), lambda t: (0, 0)),
            pl.BlockSpec((tm, 1), lambda t: (t, 0)),
            pl.BlockSpec((1, K), lambda t: (0, 0)),
        ],
        out_specs=pl.BlockSpec((tm, 1), lambda t: (t, 0)),
        out_shape=jax.ShapeDtypeStruct((M, 1), jnp.int32),
    )(r, cb, rn, cn)


def _pad_body(x_ref, out_ref):
    x = x_ref[...]
    out_ref[...] = jnp.pad(x, ((0, 0), (0, _DPAD - _D)))


def _pad_table(cbs_flat, rows_per_blk=1024):
    R = cbs_flat.shape[0]
    return pl.pallas_call(
        _pad_body,
        grid=(R // rows_per_blk,),
        in_specs=[pl.BlockSpec((rows_per_blk, _D), lambda t: (t, 0))],
        out_specs=pl.BlockSpec((rows_per_blk, _DPAD), lambda t: (t, 0)),
        out_shape=jax.ShapeDtypeStruct((R, _DPAD), jnp.float32),
    )(cbs_flat)


def _sc_gather(table, idx2d):
    V, Dp = table.shape
    B = idx2d.shape[0] * idx2d.shape[1]
    info = plsc.get_sparse_core_info()
    nw = info.num_cores * info.num_subcores
    b_per_w = B // nw
    chunks = b_per_w // 128
    mesh = plsc.VectorSubcoreMesh(core_axis_name="c", subcore_axis_name="s")

    @functools.partial(
        pl.kernel, mesh=mesh,
        out_type=jax.ShapeDtypeStruct((B, Dp), jnp.float32),
        scratch_types=[
            pltpu.VMEM((chunks, 128), jnp.int32),
            pltpu.VMEM((b_per_w, Dp), jnp.float32),
            pltpu.SemaphoreType.DMA,
        ],
    )
    def gk(table_hbm, idx_hbm, out_hbm, idx_v, rows_v, sem):
        wid = lax.axis_index("s") * info.num_cores + lax.axis_index("c")
        base = wid * b_per_w
        pltpu.sync_copy(idx_hbm.at[pl.ds(wid * chunks, chunks)], idx_v)
        cps = [pltpu.async_copy(table_hbm.at[idx_v.at[j]],
                                rows_v.at[pl.ds(j * 128, 128)], sem)
               for j in range(chunks)]
        for c in cps:
            c.wait()
        pltpu.sync_copy(rows_v, out_hbm.at[pl.ds(base, b_per_w)])

    return gk(table, idx2d)


def kernel(x, params):
    B, N, A, T = x.shape
    h = x.reshape(B, N * A, T)[:, None, :, :]
    zf, z_shape = _merged_encoder(h, params)

    cbs = params['codebooks']
    n_q = cbs.shape[0]
    table = _pad_table(cbs.reshape(n_q * _VOCAB, _D))

    total = jnp.zeros_like(zf)
    residual = zf
    for l in range(n_q):
        cb = cbs[l]
        cn = (cb ** 2).sum(-1)[None, :]
        rn = (residual ** 2).sum(-1, keepdims=True)
        idx = _nearest_code(residual, cb, rn, cn)
        gidx = idx.reshape(-1, 128) + (l * _VOCAB)
        q = _sc_gather(table, gidx)[:, :_D]
        total = total + q
        residual = residual - q

    q_st = zf + (total - zf)
    return q_st.reshape(4, z_shape[0], z_shape[1], z_shape[2])


# speedup vs baseline: 1.8920x; 1.0362x over previous
import functools

import jax
import jax.numpy as jnp
from jax import lax
from jax.experimental import pallas as pl
from jax.experimental.pallas import tpu as pltpu
from jax.experimental.pallas import tpu_sc as plsc

_K1 = [21, 15, 9, 5]
_P1 = [10, 7, 4, 2]
_K2 = [9, 7, 5, 3]
_P2 = [4, 3, 2, 1]
_GROUPS = 4
_VOCAB = 8192
_D = 200
_DPAD = 256


def _conv(x, w, b, pad, groups=1):
    y = lax.conv_general_dilated(
        x, w, window_strides=(1, 1), padding=((0, 0), (pad, pad)),
        dimension_numbers=('NCHW', 'OIHW', 'NCHW'), feature_group_count=groups)
    return y + b[None, :, None, None]


def _gn(x, g, b, groups=_GROUPS, eps=1e-5):
    B, C, H, W = x.shape
    xg = x.reshape(B, groups, C // groups, H, W)
    mu = xg.mean(axis=(2, 3, 4), keepdims=True)
    var = xg.var(axis=(2, 3, 4), keepdims=True)
    xg = (xg - mu) / jnp.sqrt(var + eps)
    xn = xg.reshape(B, C, H, W)
    return xn * g[None, :, None, None] + b[None, :, None, None]


def _pool(x, k):
    B, C, H, W = x.shape
    return x.reshape(B, C, H, W // k, k).mean(axis=-1)


def _gn_nhwc(y, g, b, groups, eps=1e-5):
    B, NA, C, T = y.shape
    yg = y.reshape(B, NA, groups, C // groups, T)
    mu = yg.mean(axis=(1, 3, 4), keepdims=True)
    var = yg.var(axis=(1, 3, 4), keepdims=True)
    yn = ((yg - mu) / jnp.sqrt(var + eps)).reshape(B, NA, C, T)
    return yn * g[None, None, :, None] + b[None, None, :, None]


def _merged_encoder(h, p):
    C = 8
    B, _, NA, T = h.shape
    K, P = _K1[0], _P1[0]
    w1 = jnp.zeros((4 * C, K), jnp.float32)
    for i in range(4):
        o = P - _P1[i]
        w1 = w1.at[i * C:(i + 1) * C, o:o + _K1[i]].set(p['c1w'][i][:, 0, 0, :])
    b1 = jnp.concatenate([p['c1b'][i] for i in range(4)])
    jj = lax.broadcasted_iota(jnp.int32, (K, T + 2 * P, T), 1)
    tt = lax.broadcasted_iota(jnp.int32, (K, T + 2 * P, T), 2)
    dd = lax.broadcasted_iota(jnp.int32, (K, T + 2 * P, T), 0)
    band = (jj - tt == dd).astype(jnp.float32)
    toep1 = jnp.einsum('od,djt->ojt', w1, band)
    xp = jnp.pad(h[:, 0], ((0, 0), (0, 0), (P, P)))
    y = jnp.einsum('bnj,ojt->bnot', xp, toep1) + b1[None, None, :, None]
    g1 = jnp.concatenate([p['g1w'][i] for i in range(4)])
    gb1 = jnp.concatenate([p['g1b'][i] for i in range(4)])
    y = jax.nn.gelu(_gn_nhwc(y, g1, gb1, 4 * _GROUPS), approximate=False)
    T2 = T // 2
    y = y.reshape(B, NA, 4 * C, T2, 2).mean(axis=-1)

    K2, P2 = _K2[0], _P2[0]
    w2 = jnp.zeros((4 * C, 4 * C, K2), jnp.float32)
    for i in range(4):
        o = P2 - _P2[i]
        w2 = w2.at[i * C:(i + 1) * C, i * C:(i + 1) * C, o:o + _K2[i]].set(
            p['c2w'][i][:, :, 0, :])
    b2 = jnp.concatenate([p['c2b'][i] for i in range(4)])
    yp = jnp.pad(y, ((0, 0), (0, 0), (0, 0), (P2, P2)))
    y2 = b2[None, None, :, None]
    for k in range(K2):
        y2 = y2 + jnp.einsum('bnct,oc->bnot', yp[..., k:k + T2], w2[:, :, k])
    g2 = jnp.concatenate([p['g2w'][i] for i in range(4)])
    gb2 = jnp.concatenate([p['g2b'][i] for i in range(4)])
    y2 = jax.nn.gelu(_gn_nhwc(y2, g2, gb2, 4 * _GROUPS), approximate=False)
    T4 = T2 // 4
    y2 = y2.reshape(B, NA, 4 * C, T4, 4).mean(axis=-1)

    z = jnp.transpose(y2.reshape(B, NA, 4, C, T4), (2, 0, 1, 4, 3))
    return z.reshape(4 * B * NA, T4 * C), (B, NA, T4 * C)


def _argmin_body(r_ref, cb_ref, rn_ref, cn_ref, out_ref):
    r = r_ref[...]
    cb = cb_ref[...]
    m = lax.dot_general(r, cb, (((1,), (1,)), ((), ())),
                        precision=lax.Precision.DEFAULT,
                        preferred_element_type=jnp.float32)
    d = (rn_ref[...] - 2.0 * m) + cn_ref[...]
    dmin = jnp.min(d, axis=1, keepdims=True)
    ids = lax.broadcasted_iota(jnp.int32, d.shape, 1)
    idx = jnp.min(jnp.where(d == dmin, ids, d.shape[1]), axis=1)
    out_ref[...] = idx[:, None]


def _nearest_code(r, cb, rn, cn, tm=256):
    M, D = r.shape
    K = cb.shape[0]
    return pl.pallas_call(
        _argmin_body,
        grid=(M // tm,),
        in_specs=[
            pl.BlockSpec((tm, D), lambda t: (t, 0)),
            pl.BlockSpec((K, D), lambda t: (0, 0)),
            pl.BlockSpec((tm, 1), lambda t: (t, 0)),
            pl.BlockSpec((1, K), lambda t: (0, 0)),
        ],
        out_specs=pl.BlockSpec((tm, 1), lambda t: (t, 0)),
        out_shape=jax.ShapeDtypeStruct((M, 1), jnp.int32),
    )(r, cb, rn, cn)


def _pad_body(x_ref, out_ref):
    x = x_ref[...]
    out_ref[...] = jnp.pad(x, ((0, 0), (0, _DPAD - _D)))


def _pad_table(cbs_flat, rows_per_blk=1024):
    R = cbs_flat.shape[0]
    return pl.pallas_call(
        _pad_body,
        grid=(R // rows_per_blk,),
        in_specs=[pl.BlockSpec((rows_per_blk, _D), lambda t: (t, 0))],
        out_specs=pl.BlockSpec((rows_per_blk, _DPAD), lambda t: (t, 0)),
        out_shape=jax.ShapeDtypeStruct((R, _DPAD), jnp.float32),
    )(cbs_flat)


def _sc_gather(table, idx2d):
    V, Dp = table.shape
    B = idx2d.shape[0] * idx2d.shape[1]
    info = plsc.get_sparse_core_info()
    nw = info.num_cores * info.num_subcores
    b_per_w = B // nw
    chunks = b_per_w // 128
    mesh = plsc.VectorSubcoreMesh(core_axis_name="c", subcore_axis_name="s")

    @functools.partial(
        pl.kernel, mesh=mesh,
        out_type=jax.ShapeDtypeStruct((B, Dp), jnp.float32),
        scratch_types=[
            pltpu.VMEM((chunks, 128), jnp.int32),
            pltpu.VMEM((b_per_w, Dp), jnp.float32),
            pltpu.SemaphoreType.DMA,
        ],
    )
    def gk(table_hbm, idx_hbm, out_hbm, idx_v, rows_v, sem):
        wid = lax.axis_index("s") * info.num_cores + lax.axis_index("c")
        base = wid * b_per_w
        pltpu.sync_copy(idx_hbm.at[pl.ds(wid * chunks, chunks)], idx_v)
        cps = [pltpu.async_copy(table_hbm.at[idx_v.at[j]],
                                rows_v.at[pl.ds(j * 128, 128)], sem)
               for j in range(chunks)]
        for c in cps:
            c.wait()
        pltpu.sync_copy(rows_v, out_hbm.at[pl.ds(base, b_per_w)])

    return gk(table, idx2d)


def kernel(x, params):
    B, N, A, T = x.shape
    h = x.reshape(B, N * A, T)[:, None, :, :]
    zf, z_shape = _merged_encoder(h, params)

    cbs = params['codebooks']
    n_q = cbs.shape[0]
    table = _pad_table(cbs.reshape(n_q * _VOCAB, _D))

    M = zf.shape[0]
    halves = [zf[:M // 2], zf[M // 2:]]
    totals = [jnp.zeros_like(hh) for hh in halves]
    residuals = list(halves)
    for l in range(n_q):
        cb = cbs[l]
        cn = (cb ** 2).sum(-1)[None, :]
        idxs = []
        for hh in range(2):
            rn = (residuals[hh] ** 2).sum(-1, keepdims=True)
            idxs.append(_nearest_code(residuals[hh], cb, rn, cn))
        for hh in range(2):
            gidx = idxs[hh].reshape(-1, 128) + (l * _VOCAB)
            q = _sc_gather(table, gidx)[:, :_D]
            totals[hh] = totals[hh] + q
            residuals[hh] = residuals[hh] - q

    total = jnp.concatenate(totals, axis=0)
    q_st = zf + (total - zf)
    return q_st.reshape(4, z_shape[0], z_shape[1], z_shape[2])


# speedup vs baseline: 1.9053x; 1.0070x over previous
import functools

import jax
import jax.numpy as jnp
from jax import lax
from jax.experimental import pallas as pl
from jax.experimental.pallas import tpu as pltpu
from jax.experimental.pallas import tpu_sc as plsc

_K1 = [21, 15, 9, 5]
_P1 = [10, 7, 4, 2]
_K2 = [9, 7, 5, 3]
_P2 = [4, 3, 2, 1]
_GROUPS = 4
_VOCAB = 8192
_D = 200
_DPAD = 256


def _conv(x, w, b, pad, groups=1):
    y = lax.conv_general_dilated(
        x, w, window_strides=(1, 1), padding=((0, 0), (pad, pad)),
        dimension_numbers=('NCHW', 'OIHW', 'NCHW'), feature_group_count=groups)
    return y + b[None, :, None, None]


def _gn(x, g, b, groups=_GROUPS, eps=1e-5):
    B, C, H, W = x.shape
    xg = x.reshape(B, groups, C // groups, H, W)
    mu = xg.mean(axis=(2, 3, 4), keepdims=True)
    var = xg.var(axis=(2, 3, 4), keepdims=True)
    xg = (xg - mu) / jnp.sqrt(var + eps)
    xn = xg.reshape(B, C, H, W)
    return xn * g[None, :, None, None] + b[None, :, None, None]


def _pool(x, k):
    B, C, H, W = x.shape
    return x.reshape(B, C, H, W // k, k).mean(axis=-1)


def _gn_nhwc(y, g, b, groups, eps=1e-5):
    B, NA, C, T = y.shape
    yg = y.reshape(B, NA, groups, C // groups, T)
    mu = yg.mean(axis=(1, 3, 4), keepdims=True)
    var = yg.var(axis=(1, 3, 4), keepdims=True)
    yn = ((yg - mu) / jnp.sqrt(var + eps)).reshape(B, NA, C, T)
    return yn * g[None, None, :, None] + b[None, None, :, None]


def _merged_encoder(h, p):
    C = 8
    B, _, NA, T = h.shape
    K, P = _K1[0], _P1[0]
    w1 = jnp.zeros((4 * C, K), jnp.float32)
    for i in range(4):
        o = P - _P1[i]
        w1 = w1.at[i * C:(i + 1) * C, o:o + _K1[i]].set(p['c1w'][i][:, 0, 0, :])
    b1 = jnp.concatenate([p['c1b'][i] for i in range(4)])
    jj = lax.broadcasted_iota(jnp.int32, (K, T + 2 * P, T), 1)
    tt = lax.broadcasted_iota(jnp.int32, (K, T + 2 * P, T), 2)
    dd = lax.broadcasted_iota(jnp.int32, (K, T + 2 * P, T), 0)
    band = (jj - tt == dd).astype(jnp.float32)
    toep1 = jnp.einsum('od,djt->ojt', w1, band)
    xp = jnp.pad(h[:, 0], ((0, 0), (0, 0), (P, P)))
    y = jnp.einsum('bnj,ojt->bnot', xp, toep1) + b1[None, None, :, None]
    g1 = jnp.concatenate([p['g1w'][i] for i in range(4)])
    gb1 = jnp.concatenate([p['g1b'][i] for i in range(4)])
    y = jax.nn.gelu(_gn_nhwc(y, g1, gb1, 4 * _GROUPS), approximate=False)
    T2 = T // 2
    y = y.reshape(B, NA, 4 * C, T2, 2).mean(axis=-1)

    K2, P2 = _K2[0], _P2[0]
    w2 = jnp.zeros((4 * C, 4 * C, K2), jnp.float32)
    for i in range(4):
        o = P2 - _P2[i]
        w2 = w2.at[i * C:(i + 1) * C, i * C:(i + 1) * C, o:o + _K2[i]].set(
            p['c2w'][i][:, :, 0, :])
    b2 = jnp.concatenate([p['c2b'][i] for i in range(4)])
    yp = jnp.pad(y, ((0, 0), (0, 0), (0, 0), (P2, P2)))
    y2 = b2[None, None, :, None]
    for k in range(K2):
        y2 = y2 + jnp.einsum('bnct,oc->bnot', yp[..., k:k + T2], w2[:, :, k])
    g2 = jnp.concatenate([p['g2w'][i] for i in range(4)])
    gb2 = jnp.concatenate([p['g2b'][i] for i in range(4)])
    y2 = jax.nn.gelu(_gn_nhwc(y2, g2, gb2, 4 * _GROUPS), approximate=False)
    T4 = T2 // 4
    y2 = y2.reshape(B, NA, 4 * C, T4, 4).mean(axis=-1)

    z = jnp.transpose(y2.reshape(B, NA, 4, C, T4), (2, 0, 1, 4, 3))
    return z.reshape(4 * B * NA, T4 * C), (B, NA, T4 * C)


def _argmin_body(r_ref, cb_ref, rn_ref, cn_ref, out_ref):
    r = r_ref[...]
    cb = cb_ref[...]
    m = lax.dot_general(r, cb, (((1,), (1,)), ((), ())),
                        precision=lax.Precision.DEFAULT,
                        preferred_element_type=jnp.float32)
    d = (rn_ref[...] - 2.0 * m) + cn_ref[...]
    idx = jnp.argmin(d, axis=1).astype(jnp.int32)
    out_ref[...] = idx[:, None]


def _nearest_code(r, cb, rn, cn, tm=256):
    M, D = r.shape
    K = cb.shape[0]
    return pl.pallas_call(
        _argmin_body,
        grid=(M // tm,),
        in_specs=[
            pl.BlockSpec((tm, D), lambda t: (t, 0)),
            pl.BlockSpec((K, D), lambda t: (0, 0)),
            pl.BlockSpec((tm, 1), lambda t: (t, 0)),
            pl.BlockSpec((1, K), lambda t: (0, 0)),
        ],
        out_specs=pl.BlockSpec((tm, 1), lambda t: (t, 0)),
        out_shape=jax.ShapeDtypeStruct((M, 1), jnp.int32),
    )(r, cb, rn, cn)


def _pad_body(x_ref, out_ref):
    x = x_ref[...]
    out_ref[...] = jnp.pad(x, ((0, 0), (0, _DPAD - _D)))


def _pad_table(cbs_flat, rows_per_blk=1024):
    R = cbs_flat.shape[0]
    return pl.pallas_call(
        _pad_body,
        grid=(R // rows_per_blk,),
        in_specs=[pl.BlockSpec((rows_per_blk, _D), lambda t: (t, 0))],
        out_specs=pl.BlockSpec((rows_per_blk, _DPAD), lambda t: (t, 0)),
        out_shape=jax.ShapeDtypeStruct((R, _DPAD), jnp.float32),
    )(cbs_flat)


def _sc_gather(table, idx2d):
    V, Dp = table.shape
    B = idx2d.shape[0] * idx2d.shape[1]
    info = plsc.get_sparse_core_info()
    nw = info.num_cores * info.num_subcores
    b_per_w = B // nw
    chunks = b_per_w // 128
    mesh = plsc.VectorSubcoreMesh(core_axis_name="c", subcore_axis_name="s")

    @functools.partial(
        pl.kernel, mesh=mesh,
        out_type=jax.ShapeDtypeStruct((B, Dp), jnp.float32),
        scratch_types=[
            pltpu.VMEM((chunks, 128), jnp.int32),
            pltpu.VMEM((b_per_w, Dp), jnp.float32),
            pltpu.SemaphoreType.DMA,
        ],
    )
    def gk(table_hbm, idx_hbm, out_hbm, idx_v, rows_v, sem):
        wid = lax.axis_index("s") * info.num_cores + lax.axis_index("c")
        base = wid * b_per_w
        pltpu.sync_copy(idx_hbm.at[pl.ds(wid * chunks, chunks)], idx_v)
        cps = [pltpu.async_copy(table_hbm.at[idx_v.at[j]],
                                rows_v.at[pl.ds(j * 128, 128)], sem)
               for j in range(chunks)]
        for c in cps:
            c.wait()
        pltpu.sync_copy(rows_v, out_hbm.at[pl.ds(base, b_per_w)])

    return gk(table, idx2d)


def kernel(x, params):
    B, N, A, T = x.shape
    h = x.reshape(B, N * A, T)[:, None, :, :]
    zf, z_shape = _merged_encoder(h, params)

    cbs = params['codebooks']
    n_q = cbs.shape[0]
    table = _pad_table(cbs.reshape(n_q * _VOCAB, _D))

    M = zf.shape[0]
    halves = [zf[:M // 2], zf[M // 2:]]
    totals = [jnp.zeros_like(hh) for hh in halves]
    residuals = list(halves)
    for l in range(n_q):
        cb = cbs[l]
        cn = (cb ** 2).sum(-1)[None, :]
        idxs = []
        for hh in range(2):
            rn = (residuals[hh] ** 2).sum(-1, keepdims=True)
            idxs.append(_nearest_code(residuals[hh], cb, rn, cn))
        for hh in range(2):
            gidx = idxs[hh].reshape(-1, 128) + (l * _VOCAB)
            q = _sc_gather(table, gidx)[:, :_D]
            totals[hh] = totals[hh] + q
            residuals[hh] = residuals[hh] - q

    total = jnp.concatenate(totals, axis=0)
    q_st = zf + (total - zf)
    return q_st.reshape(4, z_shape[0], z_shape[1], z_shape[2])
